# baseline (speedup 1.0000x reference)
import functools

import jax
import jax.numpy as jnp
from jax.experimental import pallas as pl
from jax.experimental.pallas import tpu as pltpu


def _rup128(v):
    return ((v + 127) // 128) * 128


def _conv_pool_kernel(x_ref, w_ref, b_ref, o_ref, *, img_w, g, n_conv,
                      n_out, oc):
    w = w_ref[...]
    b = b_ref[...]
    cs = img_w
    ny = n_conv - 4 * cs
    for i in range(g):
        xi = x_ref[i]
        stack = jnp.concatenate(
            [xi[:, di * 64:di * 64 + n_conv] for di in range(5)], axis=0)
        z = jnp.dot(w, stack, preferred_element_type=jnp.float32)
        acc = z[0:oc, 0:ny]
        for dj in range(1, 5):
            acc = acc + z[dj * oc:(dj + 1) * oc, dj * cs:dj * cs + ny]
        y = jnp.maximum(acc + b, 0.0)
        m01 = jnp.maximum(y[:, 0:n_out], y[:, cs:n_out + cs])
        m23 = jnp.maximum(y[:, 64:64 + n_out], y[:, 64 + cs:64 + cs + n_out])
        m = jnp.maximum(m01, m23)
        rows = [m[:, 128 * r:128 * r + 64] for r in range(n_out // 128)]
        mr = jnp.concatenate(rows, axis=1)
        z0 = jnp.zeros((mr.shape[0], o_ref.shape[2] - mr.shape[1]), mr.dtype)
        o_ref[i] = jnp.concatenate([mr, z0], axis=1).astype(o_ref.dtype)


def _conv_layer(x_frames, w_stacked, bias, *, cs, n_conv, n_out, f_in,
                f_out, g):
    nb, c_pad, _ = x_frames.shape
    oc = w_stacked.shape[0] // 5
    return pl.pallas_call(
        functools.partial(_conv_pool_kernel, img_w=cs, g=g, n_conv=n_conv,
                          n_out=n_out, oc=oc),
        out_shape=jax.ShapeDtypeStruct((nb, oc, f_out), jnp.bfloat16),
        grid=(nb // g,),
        in_specs=[
            pl.BlockSpec((g, c_pad, f_in), lambda n: (n, 0, 0)),
            pl.BlockSpec(w_stacked.shape, lambda n: (0, 0)),
            pl.BlockSpec(bias.shape, lambda n: (0, 0)),
        ],
        out_specs=pl.BlockSpec((g, oc, f_out), lambda n: (n, 0, 0)),
        compiler_params=pltpu.CompilerParams(
            dimension_semantics=("parallel",)),
    )(x_frames, w_stacked, bias)


def _fc_kernel(x_ref, w1_ref, b1_ref, w2_ref, b2_ref, w3_ref, b3_ref, o_ref):
    h = jnp.dot(x_ref[...], w1_ref[...], preferred_element_type=jnp.float32)
    h = jnp.maximum(h + b1_ref[...], 0.0).astype(jnp.bfloat16)
    h = jnp.dot(h, w2_ref[...], preferred_element_type=jnp.float32)
    h = jnp.maximum(h + b2_ref[...], 0.0).astype(jnp.bfloat16)
    o_ref[...] = (jnp.dot(h, w3_ref[...], preferred_element_type=jnp.float32)
                  + b3_ref[...])


def _fc_stack(x, w1, b1, w2, b2, w3, b3, *, bm):
    m = x.shape[0]
    bm = min(bm, m)
    n = w3.shape[1]
    return pl.pallas_call(
        _fc_kernel,
        out_shape=jax.ShapeDtypeStruct((m, n), jnp.float32),
        grid=(m // bm,),
        in_specs=[
            pl.BlockSpec((bm, x.shape[1]), lambda i: (i, 0)),
            pl.BlockSpec(w1.shape, lambda i: (0, 0)),
            pl.BlockSpec(b1.shape, lambda i: (0, 0)),
            pl.BlockSpec(w2.shape, lambda i: (0, 0)),
            pl.BlockSpec(b2.shape, lambda i: (0, 0)),
            pl.BlockSpec(w3.shape, lambda i: (0, 0)),
            pl.BlockSpec(b3.shape, lambda i: (0, 0)),
        ],
        out_specs=pl.BlockSpec((bm, n), lambda i: (i, 0)),
        compiler_params=pltpu.CompilerParams(
            dimension_semantics=("parallel",)),
    )(x, w1, b1, w2, b2, w3, b3)


def kernel(x, c1_w, c1_b, c2_w, c2_b, fc1_w, fc1_b, fc2_w, fc2_b,
           fc3_w, fc3_b):
    nb = x.shape[0]
    bf = jnp.bfloat16

    w1 = jnp.transpose(c1_w.reshape(5, 5, 8, 8),
                       (1, 2, 0, 3)).reshape(40, 40).astype(bf)
    w2 = jnp.transpose(c2_w.reshape(5, 5, 16, 8),
                       (1, 2, 0, 3)).reshape(80, 40).astype(bf)

    x1 = jnp.pad(x.reshape(nb, 3, 64 * 64),
                 ((0, 0), (0, 5), (0, 4352 - 4096))).astype(bf)

    p1 = _conv_layer(x1, w1, c1_b, cs=1, n_conv=3968, n_out=3840,
                     f_in=4352, f_out=2048, g=2)
    p2 = _conv_layer(p1, w2, c2_b, cs=2, n_conv=1792, n_out=1664,
                     f_in=2048, f_out=896, g=2)

    hflat = p2[:, :, :832].reshape(nb, 16, 13, 64)[:, :, :, 0:49:4]
    hflat = hflat.reshape(nb, 16 * 13 * 13)
    logits = _fc_stack(hflat, fc1_w.astype(bf), fc1_b,
                       fc2_w.astype(bf), fc2_b,
                       fc3_w.astype(bf), fc3_b, bm=128)
    return logits[:, :4]


# speedup vs baseline: 1.1525x; 1.1525x over previous
import functools

import jax
import jax.numpy as jnp
from jax.experimental import pallas as pl
from jax.experimental.pallas import tpu as pltpu


def _rup128(v):
    return ((v + 127) // 128) * 128


def _conv_pool_kernel(x_ref, w_ref, b_ref, o_ref, *, img_w, g, n_conv,
                      n_out, oc):
    w = w_ref[...]
    b = b_ref[...]
    cs = img_w
    ny = n_conv - 4 * cs
    for i in range(g):
        xi = x_ref[i]
        stack = jnp.concatenate(
            [xi[:, di * 64:di * 64 + n_conv] for di in range(5)], axis=0)
        z = jnp.dot(w, stack, preferred_element_type=jnp.float32)
        acc = z[0:oc, 0:ny]
        for dj in range(1, 5):
            acc = acc + z[dj * oc:(dj + 1) * oc, dj * cs:dj * cs + ny]
        y = jnp.maximum(acc + b, 0.0)
        m01 = jnp.maximum(y[:, 0:n_out], y[:, cs:n_out + cs])
        m23 = jnp.maximum(y[:, 64:64 + n_out], y[:, 64 + cs:64 + cs + n_out])
        m = jnp.maximum(m01, m23)
        rows = [m[:, 128 * r:128 * r + 64] for r in range(n_out // 128)]
        mr = jnp.concatenate(rows, axis=1)
        z0 = jnp.zeros((mr.shape[0], o_ref.shape[2] - mr.shape[1]), mr.dtype)
        o_ref[i] = jnp.concatenate([mr, z0], axis=1).astype(o_ref.dtype)


def _conv_layer(x_frames, w_stacked, bias, *, cs, n_conv, n_out, f_in,
                f_out, g):
    nb, c_pad, _ = x_frames.shape
    oc = w_stacked.shape[0] // 5
    return pl.pallas_call(
        functools.partial(_conv_pool_kernel, img_w=cs, g=g, n_conv=n_conv,
                          n_out=n_out, oc=oc),
        out_shape=jax.ShapeDtypeStruct((nb, oc, f_out), jnp.bfloat16),
        grid=(nb // g,),
        in_specs=[
            pl.BlockSpec((g, c_pad, f_in), lambda n: (n, 0, 0)),
            pl.BlockSpec(w_stacked.shape, lambda n: (0, 0)),
            pl.BlockSpec(bias.shape, lambda n: (0, 0)),
        ],
        out_specs=pl.BlockSpec((g, oc, f_out), lambda n: (n, 0, 0)),
        compiler_params=pltpu.CompilerParams(
            dimension_semantics=("parallel",)),
    )(x_frames, w_stacked, bias)


def _fc_kernel(x_ref, w1_ref, b1_ref, w2_ref, b2_ref, w3_ref, b3_ref, o_ref):
    h = jnp.dot(x_ref[...], w1_ref[...], preferred_element_type=jnp.float32)
    h = jnp.maximum(h + b1_ref[...], 0.0).astype(jnp.bfloat16)
    h = jnp.dot(h, w2_ref[...], preferred_element_type=jnp.float32)
    h = jnp.maximum(h + b2_ref[...], 0.0).astype(jnp.bfloat16)
    o_ref[...] = (jnp.dot(h, w3_ref[...], preferred_element_type=jnp.float32)
                  + b3_ref[...])


def _fc_stack(x, w1, b1, w2, b2, w3, b3, *, bm):
    m = x.shape[0]
    bm = min(bm, m)
    n = w3.shape[1]
    return pl.pallas_call(
        _fc_kernel,
        out_shape=jax.ShapeDtypeStruct((m, n), jnp.float32),
        grid=(m // bm,),
        in_specs=[
            pl.BlockSpec((bm, x.shape[1]), lambda i: (i, 0)),
            pl.BlockSpec(w1.shape, lambda i: (0, 0)),
            pl.BlockSpec(b1.shape, lambda i: (0, 0)),
            pl.BlockSpec(w2.shape, lambda i: (0, 0)),
            pl.BlockSpec(b2.shape, lambda i: (0, 0)),
            pl.BlockSpec(w3.shape, lambda i: (0, 0)),
            pl.BlockSpec(b3.shape, lambda i: (0, 0)),
        ],
        out_specs=pl.BlockSpec((bm, n), lambda i: (i, 0)),
        compiler_params=pltpu.CompilerParams(
            dimension_semantics=("parallel",)),
    )(x, w1, b1, w2, b2, w3, b3)


def kernel(x, c1_w, c1_b, c2_w, c2_b, fc1_w, fc1_b, fc2_w, fc2_b,
           fc3_w, fc3_b):
    nb = x.shape[0]
    bf = jnp.bfloat16

    w1 = jnp.transpose(c1_w.reshape(5, 5, 8, 8),
                       (1, 2, 0, 3)).reshape(40, 40).astype(bf)
    w2 = jnp.transpose(c2_w.reshape(5, 5, 16, 8),
                       (1, 2, 0, 3)).reshape(80, 40).astype(bf)

    x1 = jnp.pad(x.reshape(nb, 3, 64 * 64),
                 ((0, 0), (0, 5), (0, 4352 - 4096))).astype(bf)

    p1 = _conv_layer(x1, w1, c1_b, cs=1, n_conv=3968, n_out=3840,
                     f_in=4352, f_out=2048, g=4)
    p2 = _conv_layer(p1, w2, c2_b, cs=2, n_conv=1792, n_out=1664,
                     f_in=2048, f_out=896, g=4)

    hflat = p2[:, :, :832].reshape(nb, 16, 13, 64)[:, :, :, 0:49:4]
    hflat = hflat.reshape(nb, 16 * 13 * 13)
    logits = _fc_stack(hflat, fc1_w.astype(bf), fc1_b,
                       fc2_w.astype(bf), fc2_b,
                       fc3_w.astype(bf), fc3_b, bm=128)
    return logits[:, :4]
